# Initial kernel scaffold; baseline (speedup 1.0000x reference)
import jax
import jax.numpy as jnp
from jax.experimental import pallas as pl


def kernel(q, p):
    raise NotImplementedError("write your pallas kernel here")


# speedup vs baseline: 5.8296x; 5.8296x over previous
import jax
import jax.numpy as jnp
from jax.experimental import pallas as pl
from jax.experimental.pallas import tpu as pltpu

_N_BINS = 100
_BW = 0.1
_EPS = 1e-10
_LANES = 128


def _jsd_kernel(q_ref, p_ref, out_ref):
    nrows = q_ref.shape[0]
    n = q_ref.shape[0] * q_ref.shape[1]

    qall = q_ref[...]
    pall = p_ref[...]
    smin = jnp.minimum(jnp.min(qall), jnp.min(pall))
    smax = jnp.maximum(jnp.max(qall), jnp.max(pall))
    delta = (smax - smin) / (_N_BINS - 1)

    bidx = jax.lax.broadcasted_iota(jnp.int32, (_LANES, 1), 0).astype(jnp.float32)
    bins = smin + bidx * delta
    inv_bw = 1.0 / _BW

    def body(i, accs):
        acc_q, acc_p = accs
        qrow = q_ref[i, :].reshape(1, _LANES)
        prow = p_ref[i, :].reshape(1, _LANES)
        zq = (qrow - bins) * inv_bw
        zp = (prow - bins) * inv_bw
        acc_q = acc_q + jnp.exp(-0.5 * zq * zq)
        acc_p = acc_p + jnp.exp(-0.5 * zp * zp)
        return acc_q, acc_p

    acc0 = jnp.zeros((_LANES, _LANES), jnp.float32)
    acc_q, acc_p = jax.lax.fori_loop(0, nrows, body, (acc0, acc0))

    sum_q = jnp.sum(acc_q, axis=1, keepdims=True)
    sum_p = jnp.sum(acc_p, axis=1, keepdims=True)

    valid = bidx < _N_BINS
    pdf_q = jnp.where(valid, sum_q / n, 0.0)
    pdf_p = jnp.where(valid, sum_p / n, 0.0)
    norm_q = jnp.sum(pdf_q) + _EPS
    norm_p = jnp.sum(pdf_p) + _EPS
    qh = pdf_q / norm_q
    ph = pdf_p / norm_p

    m = 0.5 * (ph + qh)
    qh = jnp.clip(qh, 1e-45)
    ph = jnp.clip(ph, 1e-45)
    m = jnp.clip(m, 1e-45)
    lp = jnp.log(ph)
    lq = jnp.log(qh)
    lm = jnp.log(m)
    term = ph * (lp - lm) + qh * (lq - lm)
    jsd = 0.5 * jnp.sum(jnp.where(valid, term, 0.0))
    out_ref[...] = jsd.reshape(1, 1)


def kernel(q, p):
    q2 = q.reshape(-1, _LANES)
    p2 = p.reshape(-1, _LANES)
    out = pl.pallas_call(
        _jsd_kernel,
        out_shape=jax.ShapeDtypeStruct((1, 1), jnp.float32),
    )(q2, p2)
    return out[0, 0]


# speedup vs baseline: 10.7361x; 1.8416x over previous
import functools

import jax
import jax.numpy as jnp
from jax import lax
from jax.experimental import pallas as pl
from jax.experimental.pallas import tpu as pltpu
from jax.experimental.pallas import tpu_sc as plsc

_N_BINS = 100
_BW = 0.1
_EPS = 1e-10
_L = 128
_N = 262144
_F = 8192
_NC = 2
_NS = 16
_NW = _NC * _NS
_CH = _N // _NW


def _minmax_kernel(q_ref, p_ref, par_ref, minv_ref, invd_ref):
    acc_min = jnp.full((32, _L), jnp.inf, jnp.float32)
    acc_max = jnp.full((32, _L), -jnp.inf, jnp.float32)

    def body(i, accs):
        amin, amax = accs
        qb = q_ref[pl.ds(i * 32, 32), :]
        pb = p_ref[pl.ds(i * 32, 32), :]
        amin = jnp.minimum(amin, jnp.minimum(qb, pb))
        amax = jnp.maximum(amax, jnp.maximum(qb, pb))
        return amin, amax

    nit = q_ref.shape[0] // 32
    acc_min, acc_max = lax.fori_loop(0, nit, body, (acc_min, acc_max))
    smin = jnp.min(acc_min)
    smax = jnp.max(acc_max)
    rng = smax - smin
    invd = jnp.where(rng > 0, _F / rng, 0.0)
    lane = jax.lax.broadcasted_iota(jnp.int32, (1, _L), 1)
    par_ref[...] = jnp.where(lane == 0, smin,
                             jnp.where(lane == 1, smax, 0.0))
    minv_ref[...] = jnp.full((8, _L), smin, jnp.float32)
    invd_ref[...] = jnp.full((8, _L), invd, jnp.float32)


def _sc_hist_body(q_hbm, p_hbm, minv_hbm, invd_hbm, oq_hbm, op_hbm,
                  xv, hq, hp, minv_v, invd_v):
    wid = lax.axis_index("s") * _NC + lax.axis_index("c")
    base = wid * _CH
    pltpu.sync_copy(minv_hbm, minv_v)
    pltpu.sync_copy(invd_hbm, invd_v)
    minv = minv_v[0, pl.ds(0, 16)]
    invd = invd_v[0, pl.ds(0, 16)]

    zero16 = jnp.zeros((16,), jnp.float32)

    def zbody(j, c):
        hq[pl.ds(j * 16, 16)] = zero16
        hp[pl.ds(j * 16, 16)] = zero16
        return c

    lax.fori_loop(0, _F // 16, zbody, 0)

    ones = jnp.ones((16,), jnp.float32)

    def scatter_chunk(src_hbm, hist):
        pltpu.sync_copy(src_hbm.at[pl.ds(base, _CH)], xv)

        def sbody(i, c):
            x = xv[pl.ds(i * 16, 16)]
            u = (x - minv) * invd
            iu = u.astype(jnp.int32)
            iu = jnp.clip(iu, 0, _F - 1)
            plsc.addupdate_scatter(hist, [iu], ones)
            return c

        lax.fori_loop(0, _CH // 16, sbody, 0)

    scatter_chunk(q_hbm, hq)
    scatter_chunk(p_hbm, hp)
    pltpu.sync_copy(hq, oq_hbm.at[pl.ds(wid * _F, _F)])
    pltpu.sync_copy(hp, op_hbm.at[pl.ds(wid * _F, _F)])


_sc_hist = functools.partial(
    pl.kernel,
    out_type=(jax.ShapeDtypeStruct((_NW * _F,), jnp.float32),
              jax.ShapeDtypeStruct((_NW * _F,), jnp.float32)),
    mesh=plsc.VectorSubcoreMesh(core_axis_name="c", subcore_axis_name="s"),
    compiler_params=pltpu.CompilerParams(needs_layout_passes=False),
    scratch_types=[
        pltpu.VMEM((_CH,), jnp.float32),
        pltpu.VMEM((_F,), jnp.float32),
        pltpu.VMEM((_F,), jnp.float32),
        pltpu.VMEM((8, _L), jnp.float32),
        pltpu.VMEM((8, _L), jnp.float32),
    ],
)(_sc_hist_body)


def _final_kernel(hq_ref, hp_ref, par_ref, out_ref):
    smin = par_ref[0, 0]
    smax = par_ref[0, 1]
    rng = smax - smin
    delta_b = rng / (_N_BINS - 1)
    fine_d = rng / _F

    bidx = jax.lax.broadcasted_iota(jnp.int32, (_L, 1), 0).astype(jnp.float32)
    bins = smin + bidx * delta_b
    lane = jax.lax.broadcasted_iota(jnp.int32, (1, _L), 1).astype(jnp.float32)
    inv_bw = 1.0 / _BW

    def body(t, accs):
        acc_q, acc_p = accs
        hqb = jnp.sum(hq_ref[:, pl.ds(t * _L, _L)], axis=0, keepdims=True)
        hpb = jnp.sum(hp_ref[:, pl.ds(t * _L, _L)], axis=0, keepdims=True)
        cj = smin + (t * jnp.float32(_L) + lane + 0.5) * fine_d
        z = (cj - bins) * inv_bw
        g = jnp.exp(-0.5 * z * z)
        acc_q = acc_q + g * hqb
        acc_p = acc_p + g * hpb
        return acc_q, acc_p

    acc0 = jnp.zeros((_L, _L), jnp.float32)
    acc_q, acc_p = lax.fori_loop(0, _F // _L, body, (acc0, acc0))

    sum_q = jnp.sum(acc_q, axis=1, keepdims=True)
    sum_p = jnp.sum(acc_p, axis=1, keepdims=True)

    bvalid = jax.lax.broadcasted_iota(jnp.int32, (_L, 1), 0) < _N_BINS
    pdf_q = jnp.where(bvalid, sum_q / _N, 0.0)
    pdf_p = jnp.where(bvalid, sum_p / _N, 0.0)
    norm_q = jnp.sum(pdf_q) + _EPS
    norm_p = jnp.sum(pdf_p) + _EPS
    qh = pdf_q / norm_q
    ph = pdf_p / norm_p

    m = 0.5 * (ph + qh)
    qh = jnp.clip(qh, 1e-45)
    ph = jnp.clip(ph, 1e-45)
    m = jnp.clip(m, 1e-45)
    lp = jnp.log(ph)
    lq = jnp.log(qh)
    lm = jnp.log(m)
    term = ph * (lp - lm) + qh * (lq - lm)
    jsd = 0.5 * jnp.sum(jnp.where(bvalid, term, 0.0))
    out_ref[...] = jsd.reshape(1, 1)


def kernel(q, p):
    q2 = q.reshape(-1, _L)
    p2 = p.reshape(-1, _L)
    par, minv8, invd8 = pl.pallas_call(
        _minmax_kernel,
        out_shape=(
            jax.ShapeDtypeStruct((1, _L), jnp.float32),
            jax.ShapeDtypeStruct((8, _L), jnp.float32),
            jax.ShapeDtypeStruct((8, _L), jnp.float32),
        ),
    )(q2, p2)
    hq_flat, hp_flat = _sc_hist(q, p, minv8, invd8)
    hq = hq_flat.reshape(_NW, _F)
    hp = hp_flat.reshape(_NW, _F)
    out = pl.pallas_call(
        _final_kernel,
        out_shape=jax.ShapeDtypeStruct((1, 1), jnp.float32),
    )(hq, hp, par)
    return out[0, 0]


# speedup vs baseline: 11.6638x; 1.0864x over previous
import functools

import jax
import jax.numpy as jnp
from jax import lax
from jax.experimental import pallas as pl
from jax.experimental.pallas import tpu as pltpu
from jax.experimental.pallas import tpu_sc as plsc

_N_BINS = 100
_BW = 0.1
_EPS = 1e-10
_L = 128
_N = 262144
_F = 8192
_NC = 2
_NS = 16
_NW = _NC * _NS
_CH = _N // _NW


def _minmax_kernel(q_ref, p_ref, par_ref, minv_ref, invd_ref):
    acc_min = jnp.full((32, _L), jnp.inf, jnp.float32)
    acc_max = jnp.full((32, _L), -jnp.inf, jnp.float32)

    def body(i, accs):
        amin, amax = accs
        qb = q_ref[pl.ds(i * 32, 32), :]
        pb = p_ref[pl.ds(i * 32, 32), :]
        amin = jnp.minimum(amin, jnp.minimum(qb, pb))
        amax = jnp.maximum(amax, jnp.maximum(qb, pb))
        return amin, amax

    nit = q_ref.shape[0] // 32
    acc_min, acc_max = lax.fori_loop(0, nit, body, (acc_min, acc_max))
    smin = jnp.min(acc_min)
    smax = jnp.max(acc_max)
    rng = smax - smin
    invd = jnp.where(rng > 0, _F / rng, 0.0)
    lane = jax.lax.broadcasted_iota(jnp.int32, (1, _L), 1)
    par_ref[...] = jnp.where(lane == 0, smin,
                             jnp.where(lane == 1, smax, 0.0))
    minv_ref[...] = jnp.full((8, _L), smin, jnp.float32)
    invd_ref[...] = jnp.full((8, _L), invd, jnp.float32)


def _sc_hist_body(q_hbm, p_hbm, minv_hbm, invd_hbm, oq_hbm, op_hbm,
                  xq, xp, hq, hp, minv_v, invd_v, sem_q, sem_p, sem_s):
    wid = lax.axis_index("s") * _NC + lax.axis_index("c")
    base = wid * _CH
    cp_q = pltpu.async_copy(q_hbm.at[pl.ds(base, _CH)], xq, sem_q)
    cp_p = pltpu.async_copy(p_hbm.at[pl.ds(base, _CH)], xp, sem_p)
    cp_s = pltpu.async_copy(minv_hbm, minv_v, sem_s)
    cp_s2 = pltpu.async_copy(invd_hbm, invd_v, sem_s)

    zero16 = jnp.zeros((16,), jnp.float32)

    def zbody(j, c):
        b = j * 128
        for u in range(8):
            hq[pl.ds(b + u * 16, 16)] = zero16
            hp[pl.ds(b + u * 16, 16)] = zero16
        return c

    lax.fori_loop(0, _F // 128, zbody, 0)

    cp_s.wait()
    cp_s2.wait()
    minv = minv_v[0, pl.ds(0, 16)]
    invd = invd_v[0, pl.ds(0, 16)]
    ones = jnp.ones((16,), jnp.float32)

    def scatter16(src, hist, off):
        x = src[pl.ds(off, 16)]
        u = (x - minv) * invd
        iu = jnp.clip(u.astype(jnp.int32), 0, _F - 1)
        plsc.addupdate_scatter(hist, [iu], ones)

    cp_q.wait()
    cp_p.wait()

    def sbody(i, c):
        b = i * 64
        for u in range(4):
            scatter16(xq, hq, b + u * 16)
            scatter16(xp, hp, b + u * 16)
        return c

    lax.fori_loop(0, _CH // 64, sbody, 0)

    cp_oq = pltpu.async_copy(hq, oq_hbm.at[pl.ds(wid * _F, _F)], sem_q)
    cp_op = pltpu.async_copy(hp, op_hbm.at[pl.ds(wid * _F, _F)], sem_p)
    cp_oq.wait()
    cp_op.wait()


_sc_hist = functools.partial(
    pl.kernel,
    out_type=(jax.ShapeDtypeStruct((_NW * _F,), jnp.float32),
              jax.ShapeDtypeStruct((_NW * _F,), jnp.float32)),
    mesh=plsc.VectorSubcoreMesh(core_axis_name="c", subcore_axis_name="s"),
    compiler_params=pltpu.CompilerParams(needs_layout_passes=False),
    scratch_types=[
        pltpu.VMEM((_CH,), jnp.float32),
        pltpu.VMEM((_CH,), jnp.float32),
        pltpu.VMEM((_F,), jnp.float32),
        pltpu.VMEM((_F,), jnp.float32),
        pltpu.VMEM((8, _L), jnp.float32),
        pltpu.VMEM((8, _L), jnp.float32),
        pltpu.SemaphoreType.DMA,
        pltpu.SemaphoreType.DMA,
        pltpu.SemaphoreType.DMA,
    ],
)(_sc_hist_body)


def _final_kernel(hq_ref, hp_ref, par_ref, out_ref):
    smin = par_ref[0, 0]
    smax = par_ref[0, 1]
    rng = smax - smin
    delta_b = rng / (_N_BINS - 1)
    fine_d = rng / _F

    bidx = jax.lax.broadcasted_iota(jnp.int32, (_L, 1), 0).astype(jnp.float32)
    bins = smin + bidx * delta_b
    lane = jax.lax.broadcasted_iota(jnp.int32, (1, _L), 1).astype(jnp.float32)
    inv_bw = 1.0 / _BW

    def body(t, accs):
        acc_q, acc_p = accs
        hqb = jnp.sum(hq_ref[:, pl.ds(t * _L, _L)], axis=0, keepdims=True)
        hpb = jnp.sum(hp_ref[:, pl.ds(t * _L, _L)], axis=0, keepdims=True)
        cj = smin + (t * jnp.float32(_L) + lane + 0.5) * fine_d
        z = (cj - bins) * inv_bw
        g = jnp.exp(-0.5 * z * z)
        acc_q = acc_q + g * hqb
        acc_p = acc_p + g * hpb
        return acc_q, acc_p

    acc0 = jnp.zeros((_L, _L), jnp.float32)
    acc_q, acc_p = lax.fori_loop(0, _F // _L, body, (acc0, acc0))

    sum_q = jnp.sum(acc_q, axis=1, keepdims=True)
    sum_p = jnp.sum(acc_p, axis=1, keepdims=True)

    bvalid = jax.lax.broadcasted_iota(jnp.int32, (_L, 1), 0) < _N_BINS
    pdf_q = jnp.where(bvalid, sum_q / _N, 0.0)
    pdf_p = jnp.where(bvalid, sum_p / _N, 0.0)
    norm_q = jnp.sum(pdf_q) + _EPS
    norm_p = jnp.sum(pdf_p) + _EPS
    qh = pdf_q / norm_q
    ph = pdf_p / norm_p

    m = 0.5 * (ph + qh)
    qh = jnp.clip(qh, 1e-45)
    ph = jnp.clip(ph, 1e-45)
    m = jnp.clip(m, 1e-45)
    lp = jnp.log(ph)
    lq = jnp.log(qh)
    lm = jnp.log(m)
    term = ph * (lp - lm) + qh * (lq - lm)
    jsd = 0.5 * jnp.sum(jnp.where(bvalid, term, 0.0))
    out_ref[...] = jsd.reshape(1, 1)


def kernel(q, p):
    q2 = q.reshape(-1, _L)
    p2 = p.reshape(-1, _L)
    par, minv8, invd8 = pl.pallas_call(
        _minmax_kernel,
        out_shape=(
            jax.ShapeDtypeStruct((1, _L), jnp.float32),
            jax.ShapeDtypeStruct((8, _L), jnp.float32),
            jax.ShapeDtypeStruct((8, _L), jnp.float32),
        ),
    )(q2, p2)
    hq_flat, hp_flat = _sc_hist(q, p, minv8, invd8)
    hq = hq_flat.reshape(_NW, _F)
    hp = hp_flat.reshape(_NW, _F)
    out = pl.pallas_call(
        _final_kernel,
        out_shape=jax.ShapeDtypeStruct((1, 1), jnp.float32),
    )(hq, hp, par)
    return out[0, 0]


# speedup vs baseline: 15.3727x; 1.3180x over previous
import functools

import jax
import jax.numpy as jnp
from jax import lax
from jax.experimental import pallas as pl
from jax.experimental.pallas import tpu as pltpu
from jax.experimental.pallas import tpu_sc as plsc

_N_BINS = 100
_BW = 0.1
_EPS = 1e-10
_L = 128
_N = 262144
_F = 8192
_NC = 2
_NS = 16
_NW = _NC * _NS
_CH = _N // _NW


def _minmax_kernel(q_ref, p_ref, par_ref, minv_ref, invd_ref):
    acc_min = jnp.full((32, _L), jnp.inf, jnp.float32)
    acc_max = jnp.full((32, _L), -jnp.inf, jnp.float32)

    def body(i, accs):
        amin, amax = accs
        qb = q_ref[pl.ds(i * 32, 32), :]
        pb = p_ref[pl.ds(i * 32, 32), :]
        amin = jnp.minimum(amin, jnp.minimum(qb, pb))
        amax = jnp.maximum(amax, jnp.maximum(qb, pb))
        return amin, amax

    nit = q_ref.shape[0] // 32
    acc_min, acc_max = lax.fori_loop(0, nit, body, (acc_min, acc_max))
    smin = jnp.min(acc_min)
    smax = jnp.max(acc_max)
    rng = smax - smin
    invd = jnp.where(rng > 0, _F / rng, 0.0)
    lane = jax.lax.broadcasted_iota(jnp.int32, (1, _L), 1)
    par_ref[...] = jnp.where(lane == 0, smin,
                             jnp.where(lane == 1, smax, 0.0))
    minv_ref[...] = jnp.full((8, _L), smin, jnp.float32)
    invd_ref[...] = jnp.full((8, _L), invd, jnp.float32)


def _sc_hist_body(q_hbm, p_hbm, minv_hbm, invd_hbm, oq_hbm, op_hbm,
                  xq, xp, hq, hp, minv_v, invd_v, sem_q, sem_p, sem_s):
    wid = lax.axis_index("s") * _NC + lax.axis_index("c")
    base = wid * _CH
    cp_q = pltpu.async_copy(q_hbm.at[pl.ds(base, _CH)], xq, sem_q)
    cp_p = pltpu.async_copy(p_hbm.at[pl.ds(base, _CH)], xp, sem_p)
    cp_s = pltpu.async_copy(minv_hbm, minv_v, sem_s)
    cp_s2 = pltpu.async_copy(invd_hbm, invd_v, sem_s)

    zero16 = jnp.zeros((16,), jnp.float32)

    @plsc.parallel_loop(0, _F // 128, unroll=2)
    def zbody(j):
        b = j * 128
        for u in range(8):
            hq[pl.ds(b + u * 16, 16)] = zero16
            hp[pl.ds(b + u * 16, 16)] = zero16

    cp_s.wait()
    cp_s2.wait()
    minv = minv_v[0, pl.ds(0, 16)]
    invd = invd_v[0, pl.ds(0, 16)]
    ones = jnp.ones((16,), jnp.float32)

    def scatter16(src, hist, off):
        x = src[pl.ds(off, 16)]
        u = (x - minv) * invd
        iu = jnp.clip(u.astype(jnp.int32), 0, _F - 1)
        plsc.addupdate_scatter(hist, [iu], ones)

    cp_q.wait()
    cp_p.wait()

    @plsc.parallel_loop(0, _CH // 64, unroll=2)
    def sbody(i):
        b = i * 64
        for u in range(4):
            scatter16(xq, hq, b + u * 16)
            scatter16(xp, hp, b + u * 16)

    cp_oq = pltpu.async_copy(hq, oq_hbm.at[pl.ds(wid * _F, _F)], sem_q)
    cp_op = pltpu.async_copy(hp, op_hbm.at[pl.ds(wid * _F, _F)], sem_p)
    cp_oq.wait()
    cp_op.wait()


_sc_hist = functools.partial(
    pl.kernel,
    out_type=(jax.ShapeDtypeStruct((_NW * _F,), jnp.float32),
              jax.ShapeDtypeStruct((_NW * _F,), jnp.float32)),
    mesh=plsc.VectorSubcoreMesh(core_axis_name="c", subcore_axis_name="s"),
    compiler_params=pltpu.CompilerParams(needs_layout_passes=False),
    scratch_types=[
        pltpu.VMEM((_CH,), jnp.float32),
        pltpu.VMEM((_CH,), jnp.float32),
        pltpu.VMEM((_F,), jnp.float32),
        pltpu.VMEM((_F,), jnp.float32),
        pltpu.VMEM((8, _L), jnp.float32),
        pltpu.VMEM((8, _L), jnp.float32),
        pltpu.SemaphoreType.DMA,
        pltpu.SemaphoreType.DMA,
        pltpu.SemaphoreType.DMA,
    ],
)(_sc_hist_body)


def _final_kernel(hq_ref, hp_ref, par_ref, out_ref):
    smin = par_ref[0, 0]
    smax = par_ref[0, 1]
    rng = smax - smin
    delta_b = rng / (_N_BINS - 1)
    fine_d = rng / _F

    bidx = jax.lax.broadcasted_iota(jnp.int32, (_L, 1), 0).astype(jnp.float32)
    bins = smin + bidx * delta_b
    lane = jax.lax.broadcasted_iota(jnp.int32, (1, _L), 1).astype(jnp.float32)
    inv_bw = 1.0 / _BW

    def body(t, accs):
        acc_q, acc_p = accs
        hqb = jnp.sum(hq_ref[:, pl.ds(t * _L, _L)], axis=0, keepdims=True)
        hpb = jnp.sum(hp_ref[:, pl.ds(t * _L, _L)], axis=0, keepdims=True)
        cj = smin + (t * jnp.float32(_L) + lane + 0.5) * fine_d
        z = (cj - bins) * inv_bw
        g = jnp.exp(-0.5 * z * z)
        acc_q = acc_q + g * hqb
        acc_p = acc_p + g * hpb
        return acc_q, acc_p

    acc0 = jnp.zeros((_L, _L), jnp.float32)
    acc_q, acc_p = lax.fori_loop(0, _F // _L, body, (acc0, acc0))

    sum_q = jnp.sum(acc_q, axis=1, keepdims=True)
    sum_p = jnp.sum(acc_p, axis=1, keepdims=True)

    bvalid = jax.lax.broadcasted_iota(jnp.int32, (_L, 1), 0) < _N_BINS
    pdf_q = jnp.where(bvalid, sum_q / _N, 0.0)
    pdf_p = jnp.where(bvalid, sum_p / _N, 0.0)
    norm_q = jnp.sum(pdf_q) + _EPS
    norm_p = jnp.sum(pdf_p) + _EPS
    qh = pdf_q / norm_q
    ph = pdf_p / norm_p

    m = 0.5 * (ph + qh)
    qh = jnp.clip(qh, 1e-45)
    ph = jnp.clip(ph, 1e-45)
    m = jnp.clip(m, 1e-45)
    lp = jnp.log(ph)
    lq = jnp.log(qh)
    lm = jnp.log(m)
    term = ph * (lp - lm) + qh * (lq - lm)
    jsd = 0.5 * jnp.sum(jnp.where(bvalid, term, 0.0))
    out_ref[...] = jsd.reshape(1, 1)


def kernel(q, p):
    q2 = q.reshape(-1, _L)
    p2 = p.reshape(-1, _L)
    par, minv8, invd8 = pl.pallas_call(
        _minmax_kernel,
        out_shape=(
            jax.ShapeDtypeStruct((1, _L), jnp.float32),
            jax.ShapeDtypeStruct((8, _L), jnp.float32),
            jax.ShapeDtypeStruct((8, _L), jnp.float32),
        ),
    )(q2, p2)
    hq_flat, hp_flat = _sc_hist(q, p, minv8, invd8)
    hq = hq_flat.reshape(_NW, _F)
    hp = hp_flat.reshape(_NW, _F)
    out = pl.pallas_call(
        _final_kernel,
        out_shape=jax.ShapeDtypeStruct((1, 1), jnp.float32),
    )(hq, hp, par)
    return out[0, 0]
